# baseline (speedup 1.0000x reference)
import functools

import jax
import jax.numpy as jnp
from jax import lax
from jax.experimental import pallas as pl
from jax.experimental.pallas import tpu as pltpu
from jax.experimental.pallas import tpu_sc as plsc

EMB = 64
CH = 128


def _sc_gather(table, idx_t, idx_q, n_t, n_q, nw):
    info = plsc.get_sparse_core_info()
    nc, ns = info.num_cores, info.num_subcores
    assert nc * ns == nw
    kt = idx_t.shape[1]
    kq = idx_q.shape[1]
    per_t = kt * CH
    per_q = kq * CH

    mesh = plsc.VectorSubcoreMesh(core_axis_name="c", subcore_axis_name="s")

    @functools.partial(
        pl.kernel,
        out_type=(
            jax.ShapeDtypeStruct((n_t, EMB), jnp.float32),
            jax.ShapeDtypeStruct((n_q, EMB), jnp.float32),
        ),
        mesh=mesh,
        compiler_params=pltpu.CompilerParams(use_tc_tiling_on_sc=False),
        scratch_types=[
            pltpu.VMEM((kt, CH), jnp.int32),
            pltpu.VMEM((kq, CH), jnp.int32),
            pltpu.VMEM((CH, 2 * EMB), jnp.float32),
            pltpu.VMEM((CH, 2 * EMB), jnp.float32),
            pltpu.SemaphoreType.DMA,
            pltpu.SemaphoreType.DMA,
            pltpu.SemaphoreType.DMA,
            pltpu.SemaphoreType.DMA,
        ],
    )
    def k(table_h, idxt_h, idxq_h, out_t_h, out_q_h,
          idxt_v, idxq_v, rows0, rows1, gsem0, gsem1, ssem0, ssem1):
        wid = lax.axis_index("s") * nc + lax.axis_index("c")
        pltpu.sync_copy(idxt_h.at[wid], idxt_v)
        pltpu.sync_copy(idxq_h.at[wid], idxq_v)

        def run(idx_v, out_h, base, kk):
            assert kk >= 2
            rows = (rows0, rows1)
            gsem = (gsem0, gsem1)
            ssem = (ssem0, ssem1)
            pltpu.async_copy(table_h.at[idx_v.at[0]], rows[0], gsem[0])

            def step(jj, _):
                for b in range(2):
                    @pl.when(jj % 2 == b)
                    def _():
                        @pl.when(jj + 1 < kk)
                        def _():
                            @pl.when(jj >= 1)
                            def _():
                                pltpu.make_async_copy(
                                    rows[1 - b].at[:, :EMB],
                                    out_h.at[pl.ds(base, CH)],
                                    ssem[1 - b]).wait()

                            pltpu.async_copy(
                                table_h.at[idx_v.at[jj + 1]], rows[1 - b],
                                gsem[1 - b])

                        pltpu.make_async_copy(
                            table_h.at[idx_v.at[jj]], rows[b], gsem[b]).wait()
                        pltpu.async_copy(
                            rows[b].at[:, :EMB],
                            out_h.at[pl.ds(base + jj * CH, CH)],
                            ssem[b])
                return 0

            lax.fori_loop(0, kk, step, 0)
            for jj in (kk - 2, kk - 1):
                pltpu.make_async_copy(
                    rows[jj % 2].at[:, :EMB], out_h.at[pl.ds(base, CH)],
                    ssem[jj % 2]).wait()

        run(idxt_v, out_t_h, wid * per_t, kt)
        run(idxq_v, out_q_h, wid * per_q, kq)

    return k(table, idx_t, idx_q)


def _tc_body(rt_ref, rq_ref, wi_ref, bi_ref, wq_ref, bq_ref,
             w1_ref, b1_ref, w2_ref, b2_ref, out_ref, *, sp, mt, mq):
    t = jnp.tanh(rt_ref[...].reshape(mt * sp, 2 * EMB) @ wi_ref[...]
                 + bi_ref[...])
    ts = jnp.sum(t.reshape(mt, sp, 2 * EMB), axis=0) * (1.0 / mt)
    q = jnp.tanh(rq_ref[...].reshape(mq * sp, 2 * EMB) @ wq_ref[...]
                 + bq_ref[...])
    qs = jnp.sum(q.reshape(mq, sp, 2 * EMB), axis=0) * (1.0 / mq)
    pool = jnp.concatenate([ts, qs], axis=-1)
    h = jnp.maximum(pool @ w1_ref[...] + b1_ref[...], 0.0)
    out_ref[...] = h @ w2_ref[...] + b2_ref[...]


def kernel(input_x, input_x_i, input_x_q, table,
           W_t, b_t, W_i, b_i, W_q, b_q, W1, b1, W2, b2):
    del input_x, W_t, b_t
    bsz = input_x_i.shape[0]
    n_t = input_x_i.size
    n_q = input_x_q.size
    mt = n_t // bsz
    mq = n_q // bsz
    nw = 32

    idx_t = input_x_i.transpose(1, 2, 3, 0).reshape(nw, n_t // nw // CH, CH)
    idx_q = input_x_q.transpose(1, 2, 0).reshape(nw, n_q // nw // CH, CH)

    table_pad = jnp.pad(table, ((0, 0), (0, EMB)))

    rows_t, rows_q = _sc_gather(table_pad, idx_t, idx_q, n_t, n_q, nw)
    rows_t3 = rows_t.reshape(mt, bsz // 2, 2 * EMB)
    rows_q3 = rows_q.reshape(mq, bsz // 2, 2 * EMB)

    dense = W1.shape[1]
    ncls = W2.shape[1]
    zero = jnp.zeros((EMB, EMB), jnp.float32)
    wi_pack = jnp.block([[W_i, zero], [zero, W_i]])
    wq_pack = jnp.block([[W_q, zero], [zero, W_q]])
    bi_pack = jnp.concatenate([b_i, b_i]).reshape(1, 2 * EMB)
    bq_pack = jnp.concatenate([b_q, b_q]).reshape(1, 2 * EMB)
    zd = jnp.zeros((EMB, dense), jnp.float32)
    w1_pack = jnp.block([
        [W1[:EMB], zd], [zd, W1[:EMB]], [W1[EMB:], zd], [zd, W1[EMB:]]])
    b1_pack = jnp.concatenate([b1, b1]).reshape(1, 2 * dense)
    zc = jnp.zeros((dense, ncls), jnp.float32)
    w2_pack = jnp.block([[W2, zc], [zc, W2]])
    b2_pack = jnp.concatenate([b2, b2]).reshape(1, 2 * ncls)

    s = 16
    sp = s // 2
    grid = bsz // s
    out = pl.pallas_call(
        functools.partial(_tc_body, sp=sp, mt=mt, mq=mq),
        grid=(grid,),
        in_specs=[
            pl.BlockSpec((mt, sp, 2 * EMB), lambda i: (0, i, 0)),
            pl.BlockSpec((mq, sp, 2 * EMB), lambda i: (0, i, 0)),
            pl.BlockSpec((2 * EMB, 2 * EMB), lambda i: (0, 0)),
            pl.BlockSpec((1, 2 * EMB), lambda i: (0, 0)),
            pl.BlockSpec((2 * EMB, 2 * EMB), lambda i: (0, 0)),
            pl.BlockSpec((1, 2 * EMB), lambda i: (0, 0)),
            pl.BlockSpec((4 * EMB, 2 * dense), lambda i: (0, 0)),
            pl.BlockSpec((1, 2 * dense), lambda i: (0, 0)),
            pl.BlockSpec((2 * dense, 2 * ncls), lambda i: (0, 0)),
            pl.BlockSpec((1, 2 * ncls), lambda i: (0, 0)),
        ],
        out_specs=pl.BlockSpec((sp, 2 * ncls), lambda i: (i, 0)),
        out_shape=jax.ShapeDtypeStruct((bsz // 2, 2 * ncls), jnp.float32),
    )(rows_t3, rows_q3, wi_pack, bi_pack, wq_pack, bq_pack,
      w1_pack, b1_pack, w2_pack, b2_pack)
    return out.reshape(bsz, ncls)


# speedup vs baseline: 1.4861x; 1.4861x over previous
import functools

import jax
import jax.numpy as jnp
from jax import lax
from jax.experimental import pallas as pl
from jax.experimental.pallas import tpu as pltpu
from jax.experimental.pallas import tpu_sc as plsc

EMB = 64
CH = 128


def _transform_body(x_ref, wi_ref, bi_ref, wq_ref, bq_ref, e1_ref, e2_ref,
                    out_ref):
    x = x_ref[...]
    t1 = jnp.tanh(wi_ref[...] @ x + bi_ref[...])
    t2 = jnp.tanh(wq_ref[...] @ x + bq_ref[...])
    dn = (((0,), (0,)), ((), ()))
    out_ref[...] = (
        lax.dot_general(t1, e1_ref[...], dn,
                        preferred_element_type=jnp.float32)
        + lax.dot_general(t2, e2_ref[...], dn,
                          preferred_element_type=jnp.float32))


def _tc_transform(table, W_i, b_i, W_q, b_q):
    v = table.shape[0]
    table_t = table.T
    bk = 4096
    grid = pl.cdiv(v, bk)
    eye = jnp.eye(EMB, dtype=jnp.float32)
    zero = jnp.zeros((EMB, EMB), jnp.float32)
    e1 = jnp.concatenate([eye, zero], axis=1)
    e2 = jnp.concatenate([zero, eye], axis=1)
    out = pl.pallas_call(
        _transform_body,
        grid=(grid,),
        in_specs=[
            pl.BlockSpec((EMB, bk), lambda i: (0, i)),
            pl.BlockSpec((EMB, EMB), lambda i: (0, 0)),
            pl.BlockSpec((EMB, 1), lambda i: (0, 0)),
            pl.BlockSpec((EMB, EMB), lambda i: (0, 0)),
            pl.BlockSpec((EMB, 1), lambda i: (0, 0)),
            pl.BlockSpec((EMB, 2 * EMB), lambda i: (0, 0)),
            pl.BlockSpec((EMB, 2 * EMB), lambda i: (0, 0)),
        ],
        out_specs=pl.BlockSpec((bk, 2 * EMB), lambda i: (i, 0)),
        out_shape=jax.ShapeDtypeStruct((v, 2 * EMB), jnp.float32),
    )(table_t, W_i.T, b_i.reshape(EMB, 1), W_q.T, b_q.reshape(EMB, 1),
      e1, e2)
    return out.reshape(2 * v, EMB)


def _sc_gather_acc(tfm, idx_t, idx_q, slotmap, zeros, bsz, nw):
    info = plsc.get_sparse_core_info()
    nc, ns = info.num_cores, info.num_subcores
    assert nc * ns == nw
    kt = idx_t.shape[1]
    kq = idx_q.shape[1]

    mesh = plsc.VectorSubcoreMesh(core_axis_name="c", subcore_axis_name="s")

    @functools.partial(
        pl.kernel,
        out_type=(
            jax.ShapeDtypeStruct((nc, bsz, EMB), jnp.float32),
            jax.ShapeDtypeStruct((nc, bsz, EMB), jnp.float32),
        ),
        mesh=mesh,
        compiler_params=pltpu.CompilerParams(use_tc_tiling_on_sc=False),
        scratch_types=[
            pltpu.VMEM((kt, CH), jnp.int32),
            pltpu.VMEM((kq, CH), jnp.int32),
            pltpu.VMEM((8, CH), jnp.int32),
            pltpu.VMEM((CH, EMB), jnp.float32),
            pltpu.VMEM((CH, EMB), jnp.float32),
            pltpu.VMEM_SHARED((bsz, EMB), jnp.float32),
            pltpu.VMEM_SHARED((bsz, EMB), jnp.float32),
            pltpu.SemaphoreType.DMA,
            pltpu.SemaphoreType.DMA,
        ],
    )
    def k(tfm_h, idxt_h, idxq_h, slot_h, zero_h, out_t_h, out_q_h,
          idxt_v, idxq_v, slot_v, rows0, rows1, acc_t, acc_q,
          gsem0, gsem1):
        cid = lax.axis_index("c")
        sid = lax.axis_index("s")
        wid = sid * nc + cid
        pltpu.sync_copy(idxt_h.at[wid], idxt_v)
        pltpu.sync_copy(idxq_h.at[wid], idxq_v)
        pltpu.sync_copy(slot_h, slot_v)

        @pl.when(sid == 0)
        def _():
            pltpu.sync_copy(zero_h, acc_t)
            pltpu.sync_copy(zero_h, acc_q)

        plsc.subcore_barrier()

        def run(idx_v, acc, c0, kk):
            assert kk >= 2
            rows = (rows0, rows1)
            gsem = (gsem0, gsem1)
            pltpu.async_copy(tfm_h.at[idx_v.at[0]], rows[0], gsem[0])

            def step(jj, _):
                for b in range(2):
                    @pl.when(jj % 2 == b)
                    def _():
                        @pl.when(jj + 1 < kk)
                        def _():
                            pltpu.async_copy(
                                tfm_h.at[idx_v.at[jj + 1]], rows[1 - b],
                                gsem[1 - b])

                        pltpu.make_async_copy(
                            tfm_h.at[idx_v.at[jj]], rows[b], gsem[b]).wait()
                        slot = (c0 + jj) % 8
                        pltpu.sync_copy(rows[b], acc.at[slot_v.at[slot]],
                                        add=True)
                return 0

            lax.fori_loop(0, kk, step, 0)

        run(idxt_v, acc_t, wid * kt, kt)
        run(idxq_v, acc_q, wid * kq, kq)

        plsc.subcore_barrier()

        @pl.when(sid == 0)
        def _():
            pltpu.sync_copy(acc_t, out_t_h.at[cid])
            pltpu.sync_copy(acc_q, out_q_h.at[cid])

    return k(tfm, idx_t, idx_q, slotmap, zeros)


def _mlp_body(st_ref, sq_ref, w1_ref, b1_ref, w2_ref, b2_ref, out_ref,
              *, mt, mq):
    t = (st_ref[0] + st_ref[1]) * (1.0 / mt)
    q = (sq_ref[0] + sq_ref[1]) * (1.0 / mq)
    pool = jnp.concatenate([t, q], axis=-1)
    h = jnp.maximum(pool @ w1_ref[...] + b1_ref[...], 0.0)
    out_ref[...] = h @ w2_ref[...] + b2_ref[...]


def kernel(input_x, input_x_i, input_x_q, table,
           W_t, b_t, W_i, b_i, W_q, b_q, W1, b1, W2, b2):
    del input_x, W_t, b_t
    bsz = input_x_i.shape[0]
    n_t = input_x_i.size
    n_q = input_x_q.size
    mt = n_t // bsz
    mq = n_q // bsz
    nw = 32

    tfm = _tc_transform(table, W_i, b_i, W_q, b_q)

    idx_t = input_x_i.transpose(1, 2, 3, 0).reshape(nw, n_t // nw // CH, CH)
    idx_q = input_x_q.transpose(1, 2, 0).reshape(nw, n_q // nw // CH, CH)
    idx_t2 = idx_t * 2
    idx_q2 = idx_q * 2 + 1

    slotmap = (jnp.arange(8, dtype=jnp.int32)[:, None] * CH
               + jnp.arange(CH, dtype=jnp.int32)[None, :])
    zeros = jnp.zeros((bsz, EMB), jnp.float32)

    sum_t, sum_q = _sc_gather_acc(tfm, idx_t2, idx_q2, slotmap, zeros,
                                  bsz, nw)

    dense = W1.shape[1]
    ncls = W2.shape[1]
    out = pl.pallas_call(
        functools.partial(_mlp_body, mt=mt, mq=mq),
        grid=(1,),
        in_specs=[
            pl.BlockSpec((2, bsz, EMB), lambda i: (0, 0, 0)),
            pl.BlockSpec((2, bsz, EMB), lambda i: (0, 0, 0)),
            pl.BlockSpec((2 * EMB, dense), lambda i: (0, 0)),
            pl.BlockSpec((1, dense), lambda i: (0, 0)),
            pl.BlockSpec((dense, ncls), lambda i: (0, 0)),
            pl.BlockSpec((1, ncls), lambda i: (0, 0)),
        ],
        out_specs=pl.BlockSpec((bsz, ncls), lambda i: (0, 0)),
        out_shape=jax.ShapeDtypeStruct((bsz, ncls), jnp.float32),
    )(sum_t, sum_q, W1, b1.reshape(1, dense), W2, b2.reshape(1, ncls))
    return out


# speedup vs baseline: 1.9860x; 1.3363x over previous
import functools

import jax
import jax.numpy as jnp
from jax import lax
from jax.experimental import pallas as pl
from jax.experimental.pallas import tpu as pltpu
from jax.experimental.pallas import tpu_sc as plsc

EMB = 64
CH = 128


def _transform_body(x_ref, w_ref, b_ref, e_ref, out_ref):
    t = jnp.tanh(w_ref[...] @ x_ref[...] + b_ref[...])
    out_ref[...] = lax.dot_general(
        t, e_ref[...], (((0,), (0,)), ((), ())),
        preferred_element_type=jnp.float32)


def _tc_transform(table, W_i, b_i, W_q, b_q):
    v = table.shape[0]
    table_t = table.T
    bk = 8192
    grid = pl.cdiv(v, bk)
    w_stack = jnp.concatenate([W_i.T, W_q.T], axis=0)
    b_stack = jnp.concatenate([b_i, b_q]).reshape(2 * EMB, 1)
    eye = jnp.eye(EMB, dtype=jnp.float32)
    zero = jnp.zeros((EMB, EMB), jnp.float32)
    e_sel = jnp.concatenate(
        [jnp.concatenate([eye, zero], axis=1),
         jnp.concatenate([zero, eye], axis=1)], axis=0)
    out = pl.pallas_call(
        _transform_body,
        grid=(grid,),
        in_specs=[
            pl.BlockSpec((EMB, bk), lambda i: (0, i)),
            pl.BlockSpec((2 * EMB, EMB), lambda i: (0, 0)),
            pl.BlockSpec((2 * EMB, 1), lambda i: (0, 0)),
            pl.BlockSpec((2 * EMB, 2 * EMB), lambda i: (0, 0)),
        ],
        out_specs=pl.BlockSpec((bk, 2 * EMB), lambda i: (i, 0)),
        out_shape=jax.ShapeDtypeStruct((v, 2 * EMB), jnp.float32),
    )(table_t, w_stack, b_stack, e_sel)
    return out.reshape(2 * v, EMB)


def _sc_gather_acc(tfm, idx_t, idx_q, slotmap, zeros, bsz, nw):
    info = plsc.get_sparse_core_info()
    nc, ns = info.num_cores, info.num_subcores
    assert nc * ns == nw
    kt = idx_t.shape[1]
    kq = idx_q.shape[1]

    mesh = plsc.VectorSubcoreMesh(core_axis_name="c", subcore_axis_name="s")

    @functools.partial(
        pl.kernel,
        out_type=(
            jax.ShapeDtypeStruct((nc, bsz, EMB), jnp.float32),
            jax.ShapeDtypeStruct((nc, bsz, EMB), jnp.float32),
        ),
        mesh=mesh,
        compiler_params=pltpu.CompilerParams(use_tc_tiling_on_sc=False),
        scratch_types=[
            pltpu.VMEM((kt, CH), jnp.int32),
            pltpu.VMEM((kq, CH), jnp.int32),
            pltpu.VMEM((8, CH), jnp.int32),
            pltpu.VMEM((CH, EMB), jnp.float32),
            pltpu.VMEM((CH, EMB), jnp.float32),
            pltpu.VMEM_SHARED((bsz, EMB), jnp.float32),
            pltpu.VMEM_SHARED((bsz, EMB), jnp.float32),
            pltpu.SemaphoreType.DMA,
            pltpu.SemaphoreType.DMA,
        ],
    )
    def k(tfm_h, idxt_h, idxq_h, slot_h, zero_h, out_t_h, out_q_h,
          idxt_v, idxq_v, slot_v, rows0, rows1, acc_t, acc_q,
          gsem0, gsem1):
        cid = lax.axis_index("c")
        sid = lax.axis_index("s")
        wid = sid * nc + cid
        pltpu.sync_copy(idxt_h.at[wid], idxt_v)
        pltpu.sync_copy(idxq_h.at[wid], idxq_v)
        pltpu.sync_copy(slot_h, slot_v)

        @pl.when(sid == 0)
        def _():
            pltpu.sync_copy(zero_h, acc_t)
            pltpu.sync_copy(zero_h, acc_q)

        plsc.subcore_barrier()

        def run(idx_v, acc, c0, kk):
            assert kk >= 2
            rows = (rows0, rows1)
            gsem = (gsem0, gsem1)
            pltpu.async_copy(tfm_h.at[idx_v.at[0]], rows[0], gsem[0])

            def step(jj, _):
                for b in range(2):
                    @pl.when(jj % 2 == b)
                    def _():
                        @pl.when(jj + 1 < kk)
                        def _():
                            pltpu.async_copy(
                                tfm_h.at[idx_v.at[jj + 1]], rows[1 - b],
                                gsem[1 - b])

                        pltpu.make_async_copy(
                            tfm_h.at[idx_v.at[jj]], rows[b], gsem[b]).wait()
                        slot = (c0 + jj) % 8
                        pltpu.sync_copy(rows[b], acc.at[slot_v.at[slot]],
                                        add=True)
                return 0

            lax.fori_loop(0, kk, step, 0)

        run(idxt_v, acc_t, wid * kt, kt)
        run(idxq_v, acc_q, wid * kq, kq)

        plsc.subcore_barrier()

        @pl.when(sid == 0)
        def _():
            pltpu.sync_copy(acc_t, out_t_h.at[cid])
            pltpu.sync_copy(acc_q, out_q_h.at[cid])

    return k(tfm, idx_t, idx_q, slotmap, zeros)


def _mlp_body(st_ref, sq_ref, w1_ref, b1_ref, w2_ref, b2_ref, out_ref,
              *, mt, mq):
    t = (st_ref[0] + st_ref[1]) * (1.0 / mt)
    q = (sq_ref[0] + sq_ref[1]) * (1.0 / mq)
    pool = jnp.concatenate([t, q], axis=-1)
    h = jnp.maximum(pool @ w1_ref[...] + b1_ref[...], 0.0)
    out_ref[...] = h @ w2_ref[...] + b2_ref[...]


def kernel(input_x, input_x_i, input_x_q, table,
           W_t, b_t, W_i, b_i, W_q, b_q, W1, b1, W2, b2):
    del input_x, W_t, b_t
    bsz = input_x_i.shape[0]
    n_t = input_x_i.size
    n_q = input_x_q.size
    mt = n_t // bsz
    mq = n_q // bsz
    nw = 32

    tfm = _tc_transform(table, W_i, b_i, W_q, b_q)

    idx_t = input_x_i.transpose(1, 2, 3, 0).reshape(nw, n_t // nw // CH, CH)
    idx_q = input_x_q.transpose(1, 2, 0).reshape(nw, n_q // nw // CH, CH)
    idx_t2 = idx_t * 2
    idx_q2 = idx_q * 2 + 1

    slotmap = (jnp.arange(8, dtype=jnp.int32)[:, None] * CH
               + jnp.arange(CH, dtype=jnp.int32)[None, :])
    zeros = jnp.zeros((bsz, EMB), jnp.float32)

    sum_t, sum_q = _sc_gather_acc(tfm, idx_t2, idx_q2, slotmap, zeros,
                                  bsz, nw)

    dense = W1.shape[1]
    ncls = W2.shape[1]
    out = pl.pallas_call(
        functools.partial(_mlp_body, mt=mt, mq=mq),
        grid=(1,),
        in_specs=[
            pl.BlockSpec((2, bsz, EMB), lambda i: (0, 0, 0)),
            pl.BlockSpec((2, bsz, EMB), lambda i: (0, 0, 0)),
            pl.BlockSpec((2 * EMB, dense), lambda i: (0, 0)),
            pl.BlockSpec((1, dense), lambda i: (0, 0)),
            pl.BlockSpec((dense, ncls), lambda i: (0, 0)),
            pl.BlockSpec((1, ncls), lambda i: (0, 0)),
        ],
        out_specs=pl.BlockSpec((bsz, ncls), lambda i: (0, 0)),
        out_shape=jax.ShapeDtypeStruct((bsz, ncls), jnp.float32),
    )(sum_t, sum_q, W1, b1.reshape(1, dense), W2, b2.reshape(1, ncls))
    return out


# speedup vs baseline: 2.1722x; 1.0938x over previous
import functools

import jax
import jax.numpy as jnp
from jax import lax
from jax.experimental import pallas as pl
from jax.experimental.pallas import tpu as pltpu
from jax.experimental.pallas import tpu_sc as plsc

EMB = 64
CH = 128


def _transform_body(x_ref, w_ref, b_ref, e_ref, out_ref):
    t = jnp.tanh(w_ref[...] @ x_ref[...] + b_ref[...])
    out_ref[...] = lax.dot_general(
        t, e_ref[...], (((0,), (0,)), ((), ())),
        preferred_element_type=jnp.float32)


def _tc_transform(table, W_i, b_i, W_q, b_q):
    v = table.shape[0]
    table_t = table.T
    bk = 8192
    grid = pl.cdiv(v, bk)
    w_stack = jnp.concatenate([W_i.T, W_q.T], axis=0)
    b_stack = jnp.concatenate([b_i, b_q]).reshape(2 * EMB, 1)
    eye = jnp.eye(EMB, dtype=jnp.float32)
    zero = jnp.zeros((EMB, EMB), jnp.float32)
    e_sel = jnp.concatenate(
        [jnp.concatenate([eye, zero], axis=1),
         jnp.concatenate([zero, eye], axis=1)], axis=0)
    out = pl.pallas_call(
        _transform_body,
        grid=(grid,),
        in_specs=[
            pl.BlockSpec((EMB, bk), lambda i: (0, i)),
            pl.BlockSpec((2 * EMB, EMB), lambda i: (0, 0)),
            pl.BlockSpec((2 * EMB, 1), lambda i: (0, 0)),
            pl.BlockSpec((2 * EMB, 2 * EMB), lambda i: (0, 0)),
        ],
        out_specs=pl.BlockSpec((bk, 2 * EMB), lambda i: (i, 0)),
        out_shape=jax.ShapeDtypeStruct((v, 2 * EMB), jnp.float32),
    )(table_t, w_stack, b_stack, e_sel)
    return out.reshape(2 * v, EMB)


def _sc_gather_acc(tfm, idx_t, idx_q, slotmap, zeros, bsz, nw):
    info = plsc.get_sparse_core_info()
    nc, ns = info.num_cores, info.num_subcores
    assert nc * ns == nw
    kt = idx_t.shape[1]
    kq = idx_q.shape[1]

    mesh = plsc.VectorSubcoreMesh(core_axis_name="c", subcore_axis_name="s")

    @functools.partial(
        pl.kernel,
        out_type=(
            jax.ShapeDtypeStruct((nc, bsz, EMB), jnp.float32),
            jax.ShapeDtypeStruct((nc, bsz, EMB), jnp.float32),
        ),
        mesh=mesh,
        compiler_params=pltpu.CompilerParams(use_tc_tiling_on_sc=False),
        scratch_types=[
            pltpu.VMEM((kt, CH), jnp.int32),
            pltpu.VMEM((kq, CH), jnp.int32),
            pltpu.VMEM((8, CH), jnp.int32),
            pltpu.VMEM((CH, EMB), jnp.float32),
            pltpu.VMEM((CH, EMB), jnp.float32),
            pltpu.VMEM((CH, EMB), jnp.float32),
            pltpu.VMEM((CH, EMB), jnp.float32),
            pltpu.VMEM_SHARED((bsz, EMB), jnp.float32),
            pltpu.VMEM_SHARED((bsz, EMB), jnp.float32),
            pltpu.SemaphoreType.DMA,
            pltpu.SemaphoreType.DMA,
            pltpu.SemaphoreType.DMA,
            pltpu.SemaphoreType.DMA,
            pltpu.SemaphoreType.DMA,
            pltpu.SemaphoreType.DMA,
            pltpu.SemaphoreType.DMA,
            pltpu.SemaphoreType.DMA,
        ],
    )
    def k(tfm_h, idxt_h, idxq_h, slot_h, zero_h, out_t_h, out_q_h,
          idxt_v, idxq_v, slot_v, rows0, rows1, rows2, rows3, acc_t, acc_q,
          gsem0, gsem1, gsem2, gsem3, asem0, asem1, asem2, asem3):
        cid = lax.axis_index("c")
        sid = lax.axis_index("s")
        wid = sid * nc + cid
        pltpu.sync_copy(idxt_h.at[wid], idxt_v)
        pltpu.sync_copy(idxq_h.at[wid], idxq_v)
        pltpu.sync_copy(slot_h, slot_v)

        @pl.when(sid == 0)
        def _():
            pltpu.sync_copy(zero_h, acc_t)
            pltpu.sync_copy(zero_h, acc_q)

        plsc.subcore_barrier()

        def run(idx_v, acc, c0, kk):
            assert kk >= 5
            rows = (rows0, rows1, rows2, rows3)
            gsem = (gsem0, gsem1, gsem2, gsem3)
            asem = (asem0, asem1, asem2, asem3)
            for p in range(3):
                pltpu.async_copy(tfm_h.at[idx_v.at[p]], rows[p], gsem[p])

            def step(jj, _):
                for b in range(4):
                    @pl.when(jj % 4 == b)
                    def _():
                        pltpu.make_async_copy(
                            tfm_h.at[idx_v.at[jj]], rows[b], gsem[b]).wait()
                        pltpu.async_copy(
                            rows[b], acc.at[slot_v.at[(c0 + jj) % 8]],
                            asem[b], add=True)

                        @pl.when(jj + 3 < kk)
                        def _():
                            b3 = (b + 3) % 4

                            @pl.when(jj >= 1)
                            def _():
                                pltpu.make_async_copy(
                                    rows[b3],
                                    acc.at[slot_v.at[(c0 + jj - 1) % 8]],
                                    asem[b3]).wait()

                            pltpu.async_copy(
                                tfm_h.at[idx_v.at[jj + 3]], rows[b3],
                                gsem[b3])
                return 0

            lax.fori_loop(0, kk, step, 0)
            for jj in range(kk - 4, kk):
                pltpu.make_async_copy(
                    rows[jj % 4], acc.at[slot_v.at[(c0 + jj) % 8]],
                    asem[jj % 4]).wait()

        run(idxt_v, acc_t, wid * kt, kt)
        run(idxq_v, acc_q, wid * kq, kq)

        plsc.subcore_barrier()

        @pl.when(sid == 0)
        def _():
            pltpu.sync_copy(acc_t, out_t_h.at[cid])
            pltpu.sync_copy(acc_q, out_q_h.at[cid])

    return k(tfm, idx_t, idx_q, slotmap, zeros)


def _mlp_body(st_ref, sq_ref, w1_ref, b1_ref, w2_ref, b2_ref, out_ref,
              *, mt, mq):
    t = (st_ref[0] + st_ref[1]) * (1.0 / mt)
    q = (sq_ref[0] + sq_ref[1]) * (1.0 / mq)
    pool = jnp.concatenate([t, q], axis=-1)
    h = jnp.maximum(pool @ w1_ref[...] + b1_ref[...], 0.0)
    out_ref[...] = h @ w2_ref[...] + b2_ref[...]


def kernel(input_x, input_x_i, input_x_q, table,
           W_t, b_t, W_i, b_i, W_q, b_q, W1, b1, W2, b2):
    del input_x, W_t, b_t
    bsz = input_x_i.shape[0]
    n_t = input_x_i.size
    n_q = input_x_q.size
    mt = n_t // bsz
    mq = n_q // bsz
    nw = 32

    tfm = _tc_transform(table, W_i, b_i, W_q, b_q)

    idx_t = input_x_i.transpose(1, 2, 3, 0).reshape(nw, n_t // nw // CH, CH)
    idx_q = input_x_q.transpose(1, 2, 0).reshape(nw, n_q // nw // CH, CH)
    idx_t2 = idx_t * 2
    idx_q2 = idx_q * 2 + 1

    slotmap = (jnp.arange(8, dtype=jnp.int32)[:, None] * CH
               + jnp.arange(CH, dtype=jnp.int32)[None, :])
    zeros = jnp.zeros((bsz, EMB), jnp.float32)

    sum_t, sum_q = _sc_gather_acc(tfm, idx_t2, idx_q2, slotmap, zeros,
                                  bsz, nw)

    dense = W1.shape[1]
    ncls = W2.shape[1]
    out = pl.pallas_call(
        functools.partial(_mlp_body, mt=mt, mq=mq),
        grid=(1,),
        in_specs=[
            pl.BlockSpec((2, bsz, EMB), lambda i: (0, 0, 0)),
            pl.BlockSpec((2, bsz, EMB), lambda i: (0, 0, 0)),
            pl.BlockSpec((2 * EMB, dense), lambda i: (0, 0)),
            pl.BlockSpec((1, dense), lambda i: (0, 0)),
            pl.BlockSpec((dense, ncls), lambda i: (0, 0)),
            pl.BlockSpec((1, ncls), lambda i: (0, 0)),
        ],
        out_specs=pl.BlockSpec((bsz, ncls), lambda i: (0, 0)),
        out_shape=jax.ShapeDtypeStruct((bsz, ncls), jnp.float32),
    )(sum_t, sum_q, W1, b1.reshape(1, dense), W2, b2.reshape(1, ncls))
    return out
